# Initial kernel scaffold; baseline (speedup 1.0000x reference)
import jax
import jax.numpy as jnp
from jax.experimental import pallas as pl


def kernel(features, image, conv1_w, conv1_b, reg_w, reg_b, cls_w, cls_b, eye_w, eye_b):
    raise NotImplementedError("write your pallas kernel here")


# speedup vs baseline: 42.9314x; 42.9314x over previous
import functools

import numpy as np
import jax
import jax.numpy as jnp
from jax import lax
from jax.experimental import pallas as pl
from jax.experimental.pallas import tpu as pltpu
from jax.experimental.pallas import tpu_sc as plsc

_FH = _FW = 32
_IH = _IW = 512.0
_NS = 16
_L = 16
_PT = 144
_NP = _NS * _PT
_KEEP = 256
_NEG = -1e30


def _static_anchors():
    xs = _IW / _FW
    ys = _IH / _FH
    x_centers = np.arange(xs / 2.0, _IW, xs, dtype=np.float32)
    y_centers = np.arange(ys / 2.0, _IH, ys, dtype=np.float32)
    xc, yc = np.meshgrid(x_centers, y_centers, indexing='xy')
    centers = np.stack([xc.reshape(-1), yc.reshape(-1)], axis=-1)
    ratios = np.array([0.5, 1.0, 2.0], dtype=np.float32)
    scales = np.array([8.0, 16.0, 32.0], dtype=np.float32)
    s, r = np.meshgrid(scales, ratios, indexing='xy')
    s = s.reshape(-1)
    r = r.reshape(-1)
    heights = (np.sqrt(s ** 2 / r) * ys).reshape(-1)
    widths = (heights * r * xs / ys).reshape(-1)
    nc = centers.shape[0]
    na = heights.size
    centers = np.tile(centers[:, None, :], (1, na, 1))
    heights = np.tile(heights[None, :], (nc, 1))
    widths = np.tile(widths[None, :], (nc, 1))
    x_min = centers[:, :, 0] - widths / 2.0
    y_min = centers[:, :, 1] - heights / 2.0
    x_max = centers[:, :, 0] + widths / 2.0
    y_max = centers[:, :, 1] + heights / 2.0
    anchors = np.stack([x_min, y_min, x_max, y_max], axis=-1).reshape(-1, 4).astype(np.float32)
    mask = ((anchors[:, 0] >= 0.0) & (anchors[:, 1] >= 0.0)
            & (anchors[:, 2] <= _IW) & (anchors[:, 3] <= _IH))
    return anchors[mask], np.nonzero(mask)[0].astype(np.int32)


_ANCHORS, _INSIDE_IDX = _static_anchors()
_NI = _ANCHORS.shape[0]
_PER_TILE = _NI // _NS
assert _PER_TILE * _NS == _NI and _PER_TILE <= _PT

_IDX_PAD = np.zeros((_NP,), dtype=np.int32)
_ANCH_PAD = np.zeros((_NS, 4, _PT), dtype=np.float32)
_acx = (_ANCHORS[:, 0] + _ANCHORS[:, 2]) * 0.5
_acy = (_ANCHORS[:, 1] + _ANCHORS[:, 3]) * 0.5
_aw = _ANCHORS[:, 2] - _ANCHORS[:, 0]
_ah = _ANCHORS[:, 3] - _ANCHORS[:, 1]
for _t in range(_NS):
    _sl = slice(_t * _PT, _t * _PT + _PER_TILE)
    _sr = slice(_t * _PER_TILE, (_t + 1) * _PER_TILE)
    _IDX_PAD[_sl] = _INSIDE_IDX[_sr]
    _ANCH_PAD[_t, 0, :_PER_TILE] = _acx[_sr]
    _ANCH_PAD[_t, 1, :_PER_TILE] = _acy[_sr]
    _ANCH_PAD[_t, 2, :_PER_TILE] = _aw[_sr]
    _ANCH_PAD[_t, 3, :_PER_TILE] = _ah[_sr]


def _trunk_body(x_ref, w_ref, b1_ref, wh_ref, bh_ref, out_ref):
    for chunk in range(4):
        base = chunk * 256
        col = (base + lax.broadcasted_iota(jnp.int32, (256, 1), 0)) % 32
        taps = []
        for tap in range(9):
            dy, dx = tap // 3 - 1, tap % 3 - 1
            off = 33 + dy * 32 + dx + base
            a = x_ref[0, pl.ds(off, 256), :]
            if dx == -1:
                a = jnp.where(col != 0, a, 0.0)
            elif dx == 1:
                a = jnp.where(col != 31, a, 0.0)
            taps.append(a)
        big = jnp.concatenate(taps, axis=1)
        acc = jnp.dot(big, w_ref[...], preferred_element_type=jnp.float32)
        x = jnp.maximum(acc + b1_ref[...], 0.0)
        out_ref[0, pl.ds(base, 256), :] = (
            jnp.dot(x, wh_ref[...], preferred_element_type=jnp.float32)
            + bh_ref[...])


def _run_trunk(x_pad, w9, b1, wh, bh):
    return pl.pallas_call(
        _trunk_body,
        grid=(2,),
        in_specs=[
            pl.BlockSpec((1, 1090, 512), lambda b: (b, 0, 0)),
            pl.BlockSpec((4608, 512), lambda b: (0, 0)),
            pl.BlockSpec((512,), lambda b: (0,)),
            pl.BlockSpec((512, 128), lambda b: (0, 0)),
            pl.BlockSpec((128,), lambda b: (0,)),
        ],
        out_specs=pl.BlockSpec((1, 1024, 128), lambda b: (b, 0, 0)),
        out_shape=jax.ShapeDtypeStruct((2, 1024, 128), jnp.float32),
    )(x_pad, w9, b1, wh, bh)


def _nms_body(scores_hbm, deltas_hbm, idx_hbm, anch_hbm,
              keep_hbm,
              scores_v, deltas_v, idx_v, anch_v,
              x1_v, y1_v, x2_v, y2_v, ar_v, ms_v,
              d0_v, d1_v, d2_v, d3_v,
              stage_v, all16_v, keep_b, shared):
    c = lax.axis_index("c")
    s = lax.axis_index("s")
    lane = lax.broadcasted_iota(jnp.int32, (_L,), 0)

    pltpu.sync_copy(scores_hbm.at[c], scores_v)
    pltpu.sync_copy(deltas_hbm.at[c], deltas_v)
    pltpu.sync_copy(idx_hbm.at[pl.ds(s * _PT, _PT)], idx_v)
    pltpu.sync_copy(anch_hbm.at[s], anch_v)

    ngroups = _PT // _L
    for g in range(ngroups):
        rows = idx_v[pl.ds(g * _L, _L)]
        sc = plsc.load_gather(scores_v, [rows])
        d0 = plsc.load_gather(deltas_v, [jnp.full((_L,), 0, jnp.int32), rows])
        d1 = plsc.load_gather(deltas_v, [jnp.full((_L,), 1, jnp.int32), rows])
        d2 = plsc.load_gather(deltas_v, [jnp.full((_L,), 2, jnp.int32), rows])
        d3 = plsc.load_gather(deltas_v, [jnp.full((_L,), 3, jnp.int32), rows])
        acx = anch_v[0, pl.ds(g * _L, _L)]
        acy = anch_v[1, pl.ds(g * _L, _L)]
        aw = anch_v[2, pl.ds(g * _L, _L)]
        ah = anch_v[3, pl.ds(g * _L, _L)]
        cx = d0 * aw + acx
        cy = d1 * ah + acy
        w = jnp.exp(d2) * aw
        h = jnp.exp(d3) * ah
        x1 = jnp.minimum(jnp.maximum(cx - w * 0.5, 0.0), _IW)
        y1 = jnp.minimum(jnp.maximum(cy - h * 0.5, 0.0), _IH)
        x2 = jnp.minimum(jnp.maximum(cx + w * 0.5, 0.0), _IW)
        y2 = jnp.minimum(jnp.maximum(cy + h * 0.5, 0.0), _IH)
        area = (x2 - x1) * (y2 - y1)
        valid = (g * _L + lane) < _PER_TILE
        sl = pl.ds(g * _L, _L)
        x1_v[sl] = x1
        y1_v[sl] = y1
        x2_v[sl] = x2
        y2_v[sl] = y2
        ar_v[sl] = area
        ms_v[sl] = jnp.where(valid, sc, _NEG)
        d0_v[sl] = d0
        d1_v[sl] = d1
        d2_v[sl] = d2
        d3_v[sl] = d3

    def body(i, carry):
        m = ms_v[pl.ds(0, _L)]
        mi = lane
        for g in range(1, ngroups):
            v = ms_v[pl.ds(g * _L, _L)]
            upd = v > m
            m = jnp.where(upd, v, m)
            mi = jnp.where(upd, g * _L + lane, mi)
        best = jnp.max(m)
        li = jnp.min(jnp.where(m == best, mi, jnp.int32(1 << 30)))
        liv = jnp.full((_L,), li, jnp.int32)
        gidx = (s * _PT + li).astype(jnp.float32)
        fields = (best, gidx,
                  plsc.load_gather(x1_v, [liv]), plsc.load_gather(y1_v, [liv]),
                  plsc.load_gather(x2_v, [liv]), plsc.load_gather(y2_v, [liv]),
                  plsc.load_gather(ar_v, [liv]),
                  plsc.load_gather(d0_v, [liv]), plsc.load_gather(d1_v, [liv]),
                  plsc.load_gather(d2_v, [liv]), plsc.load_gather(d3_v, [liv]))
        rec = jnp.zeros((_L,), jnp.float32)
        for k, val in enumerate(fields):
            rec = jnp.where(lane == k, val, rec)
        stage_v[...] = rec
        pltpu.sync_copy(stage_v, shared.at[s])
        plsc.subcore_barrier()
        pltpu.sync_copy(shared, all16_v)
        plsc.subcore_barrier()
        zero = jnp.full((_L,), 0, jnp.int32)
        tile_scores = plsc.load_gather(all16_v, [lane, zero])
        bs = jnp.max(tile_scores)
        bj = jnp.min(jnp.where(tile_scores == bs, lane, jnp.int32(1 << 30)))
        bjv = jnp.full((_L,), bj, jnp.int32)

        def wfield(k):
            return plsc.load_gather(all16_v, [bjv, jnp.full((_L,), k, jnp.int32)])

        wg = wfield(1)
        wx1 = wfield(2)
        wy1 = wfield(3)
        wx2 = wfield(4)
        wy2 = wfield(5)
        war = wfield(6)
        for g in range(ngroups):
            sl = pl.ds(g * _L, _L)
            x1 = x1_v[sl]
            y1 = y1_v[sl]
            x2 = x2_v[sl]
            y2 = y2_v[sl]
            xx1 = jnp.maximum(x1, wx1)
            yy1 = jnp.maximum(y1, wy1)
            xx2 = jnp.minimum(x2, wx2)
            yy2 = jnp.minimum(y2, wy2)
            inter = jnp.maximum(xx2 - xx1, 0.0) * jnp.maximum(yy2 - yy1, 0.0)
            iou = inter / (war + ar_v[sl] - inter + 1e-9)
            gvec = (s * _PT + g * _L + lane).astype(jnp.float32)
            supp = (iou > 0.5) | (gvec == wg)
            ms_v[sl] = jnp.where(supp, _NEG, ms_v[sl])

        @pl.when(s == 0)
        def _():
            dmask = (lane >= 4) & (lane < 8)
            cols = jnp.where(dmask, lane + 3, 0)
            shifted = plsc.load_gather(all16_v, [bjv, cols])
            krec = jnp.where(dmask, shifted, 0.0)
            krec = jnp.where(lane == 0, (wx1 + wx2) * 0.5, krec)
            krec = jnp.where(lane == 1, (wy1 + wy2) * 0.5, krec)
            krec = jnp.where(lane == 2, wx2 - wx1, krec)
            krec = jnp.where(lane == 3, wy2 - wy1, krec)
            keep_b[i, :] = krec

        return carry

    lax.fori_loop(0, _KEEP, body, jnp.int32(0))

    @pl.when(s == 0)
    def _():
        pltpu.sync_copy(keep_b, keep_hbm.at[c])


def _run_nms(scores, deltas, idx_pad, anch_pad):
    mesh = plsc.VectorSubcoreMesh(core_axis_name="c", subcore_axis_name="s")
    f32 = jnp.float32
    kfn = pl.kernel(
        _nms_body,
        out_type=jax.ShapeDtypeStruct((2, _KEEP, _L), f32),
        mesh=mesh,
        compiler_params=pltpu.CompilerParams(needs_layout_passes=False,
                                             use_tc_tiling_on_sc=False),
        scratch_types=[
            pltpu.VMEM((9216,), f32),
            pltpu.VMEM((4, 9216), f32),
            pltpu.VMEM((_PT,), jnp.int32),
            pltpu.VMEM((4, _PT), f32),
            pltpu.VMEM((_PT,), f32),
            pltpu.VMEM((_PT,), f32),
            pltpu.VMEM((_PT,), f32),
            pltpu.VMEM((_PT,), f32),
            pltpu.VMEM((_PT,), f32),
            pltpu.VMEM((_PT,), f32),
            pltpu.VMEM((_PT,), f32),
            pltpu.VMEM((_PT,), f32),
            pltpu.VMEM((_PT,), f32),
            pltpu.VMEM((_PT,), f32),
            pltpu.VMEM((_L,), f32),
            pltpu.VMEM((_NS, _L), f32),
            pltpu.VMEM((_KEEP, _L), f32),
            pltpu.VMEM_SHARED((_NS, _L), f32),
        ],
    )
    return kfn(scores, deltas, idx_pad, anch_pad)


def kernel(features, image, conv1_w, conv1_b, reg_w, reg_b, cls_w, cls_b,
           eye_w, eye_b):
    B = features.shape[0]
    x_pad = jnp.pad(features.reshape(B, 1024, 512), ((0, 0), (33, 33), (0, 0)))
    w9 = conv1_w.reshape(9 * 512, 512)
    wh = jnp.concatenate([reg_w[0, 0], cls_w[0, 0], eye_w[0, 0]], axis=1)
    wh = jnp.pad(wh, ((0, 0), (0, 128 - 69)))
    bh = jnp.pad(jnp.concatenate([reg_b, cls_b, eye_b]), (0, 128 - 69))

    heads = _run_trunk(x_pad, w9, conv1_b, wh, bh)

    eye = heads[:, :, 45:69].reshape(B * 1024, 24)
    deltas = heads[:, :, 0:36].reshape(B, 9216, 4).transpose(0, 2, 1)
    logits = heads[:, :, 36:45].reshape(B, 9216)

    idx_pad = jnp.asarray(_IDX_PAD)
    anch_pad = jnp.asarray(_ANCH_PAD)
    keep = _run_nms(logits, deltas, idx_pad, anch_pad)
    rois = keep[:, :, 0:4]
    offs = keep[:, :, 4:8]

    anchors_b = jnp.tile(jnp.asarray(_ANCHORS)[None], (B, 1, 1))
    return rois, eye, offs, anchors_b
